# Initial kernel scaffold; baseline (speedup 1.0000x reference)
import jax
import jax.numpy as jnp
from jax.experimental import pallas as pl


def kernel(x, edge_index, weight, bias):
    raise NotImplementedError("write your pallas kernel here")


# speedup vs baseline: 41.8258x; 41.8258x over previous
import jax
import jax.numpy as jnp
from jax.experimental import pallas as pl

_N = 100
_NP = 128
_S = 128
_B = 32
_K = 30


def _fused_forward(xp_ref, w_ref, b_ref, out_ref):
    xp = xp_ref[...]
    w = w_ref[...]
    b = b_ref[...]

    nrm = jnp.sqrt(jnp.sum(xp * xp, axis=-1, keepdims=True))
    xn = xp / jnp.maximum(nrm, 1e-12)

    xn2 = xn.reshape(_NP, _B * _S)
    sim = jax.lax.dot_general(
        xn2, xn2, (((1,), (1,)), ((), ())),
        preferred_element_type=jnp.float32) * (1.0 / _B)

    row_i = jax.lax.broadcasted_iota(jnp.int32, (_NP, _NP), 0)
    col_i = jax.lax.broadcasted_iota(jnp.int32, (_NP, _NP), 1)
    diag = row_i == col_i
    eye = jnp.where(diag, 1.0, 0.0)
    valid = (row_i < _N) & (col_i < _N)

    s_masked = jnp.where(diag | ~valid, -jnp.inf, sim)
    cand = s_masked[:, None, :]
    base = s_masked[:, :, None]
    mi = jax.lax.broadcasted_iota(jnp.int32, (_NP, _NP, _NP), 1)
    mpi = jax.lax.broadcasted_iota(jnp.int32, (_NP, _NP, _NP), 2)
    beats = (cand > base) | ((cand == base) & (mpi < mi))
    cnt = jnp.sum(jnp.where(beats, 1.0, 0.0), axis=-1)
    mask = jnp.where((cnt < _K) & (~diag) & valid, 1.0, 0.0)

    mask_t = jax.lax.dot_general(
        mask, eye, (((0,), (0,)), ((), ())),
        preferred_element_type=jnp.float32)
    adj = jnp.clip(mask + mask_t, 0.0, 1.0)
    nmz = adj + eye * jnp.where(valid, 1.0, 0.0)
    common = jax.lax.dot_general(
        nmz, nmz, (((1,), (1,)), ((), ())),
        preferred_element_type=jnp.float32)
    maxc = jnp.max(common)
    denom = jnp.where(maxc > 0, maxc, 1.0)
    emask2 = adj * jnp.where(common > 1, 1.0, 0.0)
    structural = jnp.where(emask2 > 0, (common / denom) * common, 0.0)

    fused = (sim + structural) * mask
    deg = jnp.sum(fused, axis=0, keepdims=True)
    dp = deg ** -0.5
    dinv = jnp.where(jnp.isinf(dp), 0.0, dp)
    dinv_col = jax.lax.dot_general(
        eye, dinv, (((1,), (1,)), ((), ())),
        preferred_element_type=jnp.float32)
    wn = dinv_col * fused * dinv

    xw = jax.lax.dot_general(
        xp, w, (((2,), (0,)), ((), ())),
        preferred_element_type=jnp.float32)
    out = jax.lax.dot_general(
        wn, xw, (((0,), (0,)), ((), ())),
        preferred_element_type=jnp.float32)
    out_ref[...] = out + b


def kernel(x, edge_index, weight, bias):
    del edge_index
    xp = jnp.transpose(x, (2, 0, 1))
    xp = jnp.pad(xp, ((0, _NP - _N), (0, 0), (0, 0)))
    out = pl.pallas_call(
        _fused_forward,
        out_shape=jax.ShapeDtypeStruct((_NP, _B, _S), jnp.float32),
    )(xp, weight, bias.reshape(1, _S))
    return jnp.transpose(out[:_N], (1, 2, 0))


# speedup vs baseline: 83.5057x; 1.9965x over previous
import jax
import jax.numpy as jnp
from jax.experimental import pallas as pl

_N = 100
_NP = 128
_S = 128
_B = 32
_K = 30


def _fused_forward(y_ref, w_ref, b_ref, out_ref):
    y2 = y_ref[...]
    w = w_ref[...]
    bias = b_ref[...]

    y3 = y2.reshape(_B, _S, _NP)
    nr2 = jnp.sum(y3 * y3, axis=1, keepdims=True)
    nrm = jnp.sqrt(nr2)
    yn3 = y3 / jnp.maximum(nrm, 1e-12)
    yn2 = yn3.reshape(_B * _S, _NP)

    sim = jax.lax.dot_general(
        yn2, yn2, (((0,), (0,)), ((), ())),
        preferred_element_type=jnp.float32) * (1.0 / _B)

    row_i = jax.lax.broadcasted_iota(jnp.int32, (_NP, _NP), 0)
    col_i = jax.lax.broadcasted_iota(jnp.int32, (_NP, _NP), 1)
    diag = row_i == col_i
    eye = jnp.where(diag, 1.0, 0.0)
    ones_col = jnp.full((_NP, 1), 1.0, dtype=jnp.float32)

    s_masked = jnp.where(diag | (col_i >= _N), -jnp.inf, sim)
    bits = jax.lax.bitcast_convert_type(s_masked, jnp.int32)
    key = bits ^ ((bits >> 31) & jnp.int32(0x7FFFFFFF))

    def _bs_step(_, carry):
        lo, hi = carry
        mid = (lo >> 1) + (hi >> 1) + (lo & hi & 1)
        midb = jax.lax.broadcast_in_dim(mid, (_NP, _NP), (0, 1))
        ge = jnp.where(key >= midb, 1.0, 0.0)
        cnt = jax.lax.dot_general(
            ge, ones_col, (((1,), (0,)), ((), ())),
            preferred_element_type=jnp.float32)
        take = cnt >= float(_K)
        return jnp.where(take, mid, lo), jnp.where(take, hi, mid)

    lo0 = jnp.full((_NP, 1), jnp.iinfo(jnp.int32).min, dtype=jnp.int32)
    hi0 = jnp.full((_NP, 1), jnp.iinfo(jnp.int32).max, dtype=jnp.int32)
    x_thr, _ = jax.lax.fori_loop(0, 32, _bs_step, (lo0, hi0))

    xb = jax.lax.broadcast_in_dim(x_thr, (_NP, _NP), (0, 1))
    gt = jnp.where(key > xb, 1.0, 0.0)
    eq = jnp.where(key == xb, 1.0, 0.0)
    g_row = jax.lax.dot_general(
        gt, ones_col, (((1,), (0,)), ((), ())),
        preferred_element_type=jnp.float32)
    upper = jnp.where(row_i < col_i, 1.0, 0.0)
    prefix = jax.lax.dot_general(
        eq, upper, (((1,), (0,)), ((), ())),
        preferred_element_type=jnp.float32)
    gb = jax.lax.broadcast_in_dim(g_row, (_NP, _NP), (0, 1))
    sel_eq = eq * jnp.where(gb + prefix < float(_K), 1.0, 0.0)
    rowvalid = jnp.where(row_i < _N, 1.0, 0.0)
    mask = (gt + sel_eq) * rowvalid

    mask_t = jax.lax.dot_general(
        mask, eye, (((0,), (0,)), ((), ())),
        preferred_element_type=jnp.float32)
    adj = jnp.clip(mask + mask_t, 0.0, 1.0)
    nmz = adj + eye * rowvalid * jnp.where(col_i < _N, 1.0, 0.0)
    common = jax.lax.dot_general(
        nmz, nmz, (((1,), (1,)), ((), ())),
        preferred_element_type=jnp.float32)
    maxc = jnp.max(common)
    denom = jnp.where(maxc > 0, maxc, 1.0)
    emask2 = adj * jnp.where(common > 1, 1.0, 0.0)
    structural = jnp.where(emask2 > 0, (common / denom) * common, 0.0)

    fused = (sim + structural) * mask
    deg = jnp.sum(fused, axis=0, keepdims=True)
    dp = deg ** -0.5
    dinv = jnp.where(jnp.isinf(dp), 0.0, dp)
    dinv_col = jax.lax.dot_general(
        eye, dinv, (((1,), (1,)), ((), ())),
        preferred_element_type=jnp.float32)
    wn = dinv_col * fused * dinv

    p2 = jax.lax.dot_general(
        y2, wn, (((1,), (0,)), ((), ())),
        preferred_element_type=jnp.float32)
    p3 = p2.reshape(_B, _S, _NP)
    w3 = jnp.broadcast_to(w[None, :, :], (_B, _S, _S))
    out = jax.lax.dot_general(
        w3, p3, (((1,), (1,)), ((0,), (0,))),
        preferred_element_type=jnp.float32)
    out_ref[...] = out + bias.reshape(1, _S, 1)


def kernel(x, edge_index, weight, bias):
    del edge_index
    y = jnp.pad(x, ((0, 0), (0, 0), (0, _NP - _N))).reshape(_B * _S, _NP)
    out = pl.pallas_call(
        _fused_forward,
        out_shape=jax.ShapeDtypeStruct((_B, _S, _NP), jnp.float32),
    )(y, weight, bias.reshape(1, _S))
    return out[:, :, :_N]


# speedup vs baseline: 85.2922x; 1.0214x over previous
import jax
import jax.numpy as jnp
from jax.experimental import pallas as pl

_N = 100
_NP = 128
_S = 128
_B = 32
_K = 30


def _fused_forward(x_ref, w_ref, b_ref, out_ref):
    xr = x_ref[...]
    w = w_ref[...]
    bias = b_ref[...]

    y3 = jnp.pad(xr, ((0, 0), (0, 0), (0, _NP - _N)))
    y2 = y3.reshape(_B * _S, _NP)

    nr2 = jnp.sum(y3 * y3, axis=1, keepdims=True)
    nrm = jnp.sqrt(nr2)
    yn3 = y3 / jnp.maximum(nrm, 1e-12)
    yn2 = yn3.reshape(_B * _S, _NP)

    sim = jax.lax.dot_general(
        yn2, yn2, (((0,), (0,)), ((), ())),
        preferred_element_type=jnp.float32) * (1.0 / _B)

    row_i = jax.lax.broadcasted_iota(jnp.int32, (_NP, _NP), 0)
    col_i = jax.lax.broadcasted_iota(jnp.int32, (_NP, _NP), 1)
    diag = row_i == col_i
    eye = jnp.where(diag, 1.0, 0.0)
    ones_col = jnp.full((_NP, 1), 1.0, dtype=jnp.float32)

    s_masked = jnp.where(diag | (col_i >= _N), -jnp.inf, sim)
    bits = jax.lax.bitcast_convert_type(s_masked, jnp.int32)
    key = bits ^ ((bits >> 31) & jnp.int32(0x7FFFFFFF))

    def _bs_step(_, carry):
        lo, hi = carry
        mid = (lo >> 1) + (hi >> 1) + (lo & hi & 1)
        midb = jax.lax.broadcast_in_dim(mid, (_NP, _NP), (0, 1))
        ge = jnp.where(key >= midb, 1.0, 0.0)
        cnt = jax.lax.dot_general(
            ge, ones_col, (((1,), (0,)), ((), ())),
            preferred_element_type=jnp.float32)
        take = cnt >= float(_K)
        return jnp.where(take, mid, lo), jnp.where(take, hi, mid)

    lo0 = jnp.full((_NP, 1), jnp.iinfo(jnp.int32).min, dtype=jnp.int32)
    hi0 = jnp.full((_NP, 1), jnp.iinfo(jnp.int32).max, dtype=jnp.int32)
    x_thr, _ = jax.lax.fori_loop(0, 32, _bs_step, (lo0, hi0))

    xb = jax.lax.broadcast_in_dim(x_thr, (_NP, _NP), (0, 1))
    gt = jnp.where(key > xb, 1.0, 0.0)
    eq = jnp.where(key == xb, 1.0, 0.0)
    g_row = jax.lax.dot_general(
        gt, ones_col, (((1,), (0,)), ((), ())),
        preferred_element_type=jnp.float32)
    upper = jnp.where(row_i < col_i, 1.0, 0.0)
    prefix = jax.lax.dot_general(
        eq, upper, (((1,), (0,)), ((), ())),
        preferred_element_type=jnp.float32)
    gb = jax.lax.broadcast_in_dim(g_row, (_NP, _NP), (0, 1))
    sel_eq = eq * jnp.where(gb + prefix < float(_K), 1.0, 0.0)
    rowvalid = jnp.where(row_i < _N, 1.0, 0.0)
    mask = (gt + sel_eq) * rowvalid

    mask_t = jax.lax.dot_general(
        mask, eye, (((0,), (0,)), ((), ())),
        preferred_element_type=jnp.float32)
    adj = jnp.clip(mask + mask_t, 0.0, 1.0)
    nmz = adj + eye * rowvalid * jnp.where(col_i < _N, 1.0, 0.0)
    common = jax.lax.dot_general(
        nmz, nmz, (((1,), (1,)), ((), ())),
        preferred_element_type=jnp.float32)
    maxc = jnp.max(common)
    denom = jnp.where(maxc > 0, maxc, 1.0)
    emask2 = adj * jnp.where(common > 1, 1.0, 0.0)
    structural = jnp.where(emask2 > 0, (common / denom) * common, 0.0)

    fused = (sim + structural) * mask
    deg = jnp.sum(fused, axis=0, keepdims=True)
    dp = deg ** -0.5
    dinv = jnp.where(jnp.isinf(dp), 0.0, dp)
    dinv_col = jax.lax.dot_general(
        eye, dinv, (((1,), (1,)), ((), ())),
        preferred_element_type=jnp.float32)
    wn = dinv_col * fused * dinv

    p2 = jax.lax.dot_general(
        y2, wn, (((1,), (0,)), ((), ())),
        preferred_element_type=jnp.float32)
    p3 = p2.reshape(_B, _S, _NP)
    w3 = jnp.broadcast_to(w[None, :, :], (_B, _S, _S))
    out = jax.lax.dot_general(
        w3, p3, (((1,), (1,)), ((0,), (0,))),
        preferred_element_type=jnp.float32)
    out_ref[...] = (out + bias.reshape(1, _S, 1))[:, :, :_N]


def kernel(x, edge_index, weight, bias):
    del edge_index
    return pl.pallas_call(
        _fused_forward,
        out_shape=jax.ShapeDtypeStruct((_B, _S, _N), jnp.float32),
    )(x, weight, bias.reshape(1, _S))


# speedup vs baseline: 107.7084x; 1.2628x over previous
import jax
import jax.numpy as jnp
from jax.experimental import pallas as pl

_N = 100
_NP = 128
_S = 128
_B = 32
_K = 30


def _fused_forward(x_ref, w_ref, b_ref, out_ref):
    xr = x_ref[...]
    w = w_ref[...]
    bias = b_ref[...]

    y3 = jnp.pad(xr, ((0, 0), (0, 0), (0, _NP - _N)))
    y2 = y3.reshape(_B * _S, _NP)

    nr2 = jnp.sum(y3 * y3, axis=1, keepdims=True)
    nrm = jnp.sqrt(nr2)
    yn3 = y3 / jnp.maximum(nrm, 1e-12)
    yn2 = yn3.reshape(_B * _S, _NP)

    sim = jax.lax.dot_general(
        yn2, yn2, (((0,), (0,)), ((), ())),
        preferred_element_type=jnp.float32) * (1.0 / _B)

    row_i = jax.lax.broadcasted_iota(jnp.int32, (_NP, _NP), 0)
    col_i = jax.lax.broadcasted_iota(jnp.int32, (_NP, _NP), 1)
    diag = row_i == col_i
    eye = jnp.where(diag, 1.0, 0.0)
    ones_col = jnp.full((_NP, 1), 1.0, dtype=jnp.float32)

    s_masked = jnp.where(diag | (col_i >= _N), -jnp.inf, sim)
    bits = jax.lax.bitcast_convert_type(s_masked, jnp.int32)
    key = bits ^ ((bits >> 31) & jnp.int32(0x7FFFFFFF))

    p_pre = jnp.full((_NP, 1), jnp.iinfo(jnp.int32).min, dtype=jnp.int32)
    for t, wbits in [(30, 2)] + [(27 - 3 * i, 3) for i in range(10)]:
        pb = jax.lax.broadcast_in_dim(p_pre, (_NP, _NP), (0, 1))
        jstar = jnp.zeros((_NP, 1), dtype=jnp.float32)
        for j in range(1, 1 << wbits):
            step = (j << t) & 0xFFFFFFFF
            step -= 1 << 32 if step >= (1 << 31) else 0
            gej = jnp.where(key >= pb + jnp.int32(step), 1.0, 0.0)
            cntj = jax.lax.dot_general(
                gej, ones_col, (((1,), (0,)), ((), ())),
                preferred_element_type=jnp.float32)
            jstar = jstar + jnp.where(cntj >= float(_K), 1.0, 0.0)
        p_pre = p_pre + (jstar.astype(jnp.int32) << t)
    x_thr = p_pre

    xb = jax.lax.broadcast_in_dim(x_thr, (_NP, _NP), (0, 1))
    gt = jnp.where(key > xb, 1.0, 0.0)
    eq = jnp.where(key == xb, 1.0, 0.0)
    g_row = jax.lax.dot_general(
        gt, ones_col, (((1,), (0,)), ((), ())),
        preferred_element_type=jnp.float32)
    upper = jnp.where(row_i < col_i, 1.0, 0.0)
    prefix = jax.lax.dot_general(
        eq, upper, (((1,), (0,)), ((), ())),
        preferred_element_type=jnp.float32)
    gb = jax.lax.broadcast_in_dim(g_row, (_NP, _NP), (0, 1))
    sel_eq = eq * jnp.where(gb + prefix < float(_K), 1.0, 0.0)
    rowvalid = jnp.where(row_i < _N, 1.0, 0.0)
    mask = (gt + sel_eq) * rowvalid

    mask_t = jax.lax.dot_general(
        mask, eye, (((0,), (0,)), ((), ())),
        preferred_element_type=jnp.float32)
    adj = jnp.clip(mask + mask_t, 0.0, 1.0)
    nmz = adj + eye * rowvalid * jnp.where(col_i < _N, 1.0, 0.0)
    common = jax.lax.dot_general(
        nmz, nmz, (((1,), (1,)), ((), ())),
        preferred_element_type=jnp.float32)
    maxc = jnp.max(common)
    denom = jnp.where(maxc > 0, maxc, 1.0)
    emask2 = adj * jnp.where(common > 1, 1.0, 0.0)
    structural = jnp.where(emask2 > 0, (common / denom) * common, 0.0)

    fused = (sim + structural) * mask
    deg = jnp.sum(fused, axis=0, keepdims=True)
    dp = deg ** -0.5
    dinv = jnp.where(jnp.isinf(dp), 0.0, dp)
    dinv_col = jax.lax.dot_general(
        eye, dinv, (((1,), (1,)), ((), ())),
        preferred_element_type=jnp.float32)
    wn = dinv_col * fused * dinv

    p2 = jax.lax.dot_general(
        y2, wn, (((1,), (0,)), ((), ())),
        preferred_element_type=jnp.float32)
    p3 = p2.reshape(_B, _S, _NP)
    w3 = jnp.broadcast_to(w[None, :, :], (_B, _S, _S))
    out = jax.lax.dot_general(
        w3, p3, (((1,), (1,)), ((0,), (0,))),
        preferred_element_type=jnp.float32)
    out_ref[...] = (out + bias.reshape(1, _S, 1))[:, :, :_N]


def kernel(x, edge_index, weight, bias):
    del edge_index
    return pl.pallas_call(
        _fused_forward,
        out_shape=jax.ShapeDtypeStruct((_B, _S, _N), jnp.float32),
    )(x, weight, bias.reshape(1, _S))


# speedup vs baseline: 107.8018x; 1.0009x over previous
import jax
import jax.numpy as jnp
from jax.experimental import pallas as pl

_N = 100
_NP = 128
_S = 128
_B = 32
_K = 30


def _fused_forward(x_ref, w_ref, b_ref, out_ref):
    xr = x_ref[...]
    w = w_ref[...]
    bias = b_ref[...]

    y3 = jnp.pad(xr, ((0, 0), (0, 0), (0, _NP - _N)))
    y2 = y3.reshape(_B * _S, _NP)

    nr2 = jnp.sum(y3 * y3, axis=1, keepdims=True)
    nrm = jnp.sqrt(nr2)
    yn3 = y3 / jnp.maximum(nrm, 1e-12)
    yn2 = yn3.reshape(_B * _S, _NP)

    sim = jax.lax.dot_general(
        yn2, yn2, (((0,), (0,)), ((), ())),
        preferred_element_type=jnp.float32) * (1.0 / _B)

    row_i = jax.lax.broadcasted_iota(jnp.int32, (_NP, _NP), 0)
    col_i = jax.lax.broadcasted_iota(jnp.int32, (_NP, _NP), 1)
    diag = row_i == col_i
    eye = jnp.where(diag, 1.0, 0.0)
    ones_col = jnp.full((_NP, 1), 1.0, dtype=jnp.float32)

    s_masked = jnp.where(diag | (col_i >= _N), -jnp.inf, sim)
    bits = jax.lax.bitcast_convert_type(s_masked, jnp.int32)
    key = bits ^ ((bits >> 31) & jnp.int32(0x7FFFFFFF))

    p_pre = jnp.full((_NP, 1), jnp.iinfo(jnp.int32).min, dtype=jnp.int32)
    for t, wbits in [(30, 2)] + [(27 - 3 * i, 3) for i in range(10)]:
        pb = jax.lax.broadcast_in_dim(p_pre, (_NP, _NP), (0, 1))
        jstar = jnp.zeros((_NP, 1), dtype=jnp.float32)
        for j in range(1, 1 << wbits):
            step = (j << t) & 0xFFFFFFFF
            step -= 1 << 32 if step >= (1 << 31) else 0
            gej = jnp.where(key >= pb + jnp.int32(step), 1.0, 0.0)
            cntj = jax.lax.dot_general(
                gej, ones_col, (((1,), (0,)), ((), ())),
                preferred_element_type=jnp.float32)
            jstar = jstar + jnp.where(cntj >= float(_K), 1.0, 0.0)
        p_pre = p_pre + (jstar.astype(jnp.int32) << t)
    x_thr = p_pre

    xb = jax.lax.broadcast_in_dim(x_thr, (_NP, _NP), (0, 1))
    gt = jnp.where(key > xb, 1.0, 0.0)
    eq = jnp.where(key == xb, 1.0, 0.0)
    g_row = jax.lax.dot_general(
        gt, ones_col, (((1,), (0,)), ((), ())),
        preferred_element_type=jnp.float32)
    upper = jnp.where(row_i < col_i, 1.0, 0.0)
    prefix = jax.lax.dot_general(
        eq, upper, (((1,), (0,)), ((), ())),
        preferred_element_type=jnp.float32)
    gb = jax.lax.broadcast_in_dim(g_row, (_NP, _NP), (0, 1))
    sel_eq = eq * jnp.where(gb + prefix < float(_K), 1.0, 0.0)
    rowvalid = jnp.where(row_i < _N, 1.0, 0.0)
    mask = (gt + sel_eq) * rowvalid

    mask_t = jax.lax.dot_general(
        mask, eye, (((0,), (0,)), ((), ())),
        preferred_element_type=jnp.float32)
    adj = jnp.clip(mask + mask_t, 0.0, 1.0)
    nmz = adj + eye * rowvalid * jnp.where(col_i < _N, 1.0, 0.0)
    common = jax.lax.dot_general(
        nmz, nmz, (((1,), (1,)), ((), ())),
        preferred_element_type=jnp.float32)
    maxc = jnp.max(common)
    denom = jnp.where(maxc > 0, maxc, 1.0)
    emask2 = adj * jnp.where(common > 1, 1.0, 0.0)
    structural = jnp.where(emask2 > 0, (common / denom) * common, 0.0)

    fused = (sim + structural) * mask
    deg = jnp.sum(fused, axis=0, keepdims=True)
    dp = deg ** -0.5
    dinv = jnp.where(jnp.isinf(dp), 0.0, dp)
    dinv_col = jax.lax.dot_general(
        eye, dinv, (((1,), (1,)), ((), ())),
        preferred_element_type=jnp.float32)
    wn = dinv_col * fused * dinv

    p2 = jax.lax.dot_general(
        y2, wn, (((1,), (0,)), ((), ())),
        preferred_element_type=jnp.float32)
    p3 = p2.reshape(_B, _S, _NP)
    bias_col = jax.lax.dot_general(
        eye[:_S, :_S], bias, (((1,), (1,)), ((), ())),
        preferred_element_type=jnp.float32)
    for b in range(_B):
        outb = jax.lax.dot_general(
            w, p3[b], (((0,), (0,)), ((), ())),
            preferred_element_type=jnp.float32)
        out_ref[b] = (outb + bias_col)[:, :_N]


def kernel(x, edge_index, weight, bias):
    del edge_index
    return pl.pallas_call(
        _fused_forward,
        out_shape=jax.ShapeDtypeStruct((_B, _S, _N), jnp.float32),
    )(x, weight, bias.reshape(1, _S))


# speedup vs baseline: 108.3467x; 1.0051x over previous
import jax
import jax.numpy as jnp
from jax.experimental import pallas as pl

_N = 100
_NP = 128
_S = 128
_B = 32
_K = 30


def _fused_forward(x_ref, w_ref, b_ref, out_ref):
    xr = x_ref[...]
    w = w_ref[...]
    bias = b_ref[...]

    y3 = jnp.pad(xr, ((0, 0), (0, 0), (0, _NP - _N)))
    y2 = y3.reshape(_B * _S, _NP)

    nr2 = jnp.sum(y3 * y3, axis=1, keepdims=True)
    nrm = jnp.sqrt(nr2)
    yn3 = y3 / jnp.maximum(nrm, 1e-12)
    yn2 = yn3.reshape(_B * _S, _NP)

    sim = jax.lax.dot_general(
        yn2, yn2, (((0,), (0,)), ((), ())),
        preferred_element_type=jnp.float32) * (1.0 / _B)

    row_i = jax.lax.broadcasted_iota(jnp.int32, (_NP, _NP), 0)
    col_i = jax.lax.broadcasted_iota(jnp.int32, (_NP, _NP), 1)
    diag = row_i == col_i
    eye = jnp.where(diag, 1.0, 0.0)
    ones_col = jnp.full((_NP, 1), 1.0, dtype=jnp.float32)

    s_masked = jnp.where(diag | (col_i >= _N), -jnp.inf, sim)
    bits = jax.lax.bitcast_convert_type(s_masked, jnp.int32)
    key = bits ^ ((bits >> 31) & jnp.int32(0x7FFFFFFF))

    p_pre = jnp.full((_NP, 1), jnp.iinfo(jnp.int32).min, dtype=jnp.int32)
    for t, wbits in [(28 - 4 * i, 4) for i in range(8)]:
        pb = jax.lax.broadcast_in_dim(p_pre, (_NP, _NP), (0, 1))
        jstar = jnp.zeros((_NP, 1), dtype=jnp.float32)
        for j in range(1, 1 << wbits):
            step = (j << t) & 0xFFFFFFFF
            step -= 1 << 32 if step >= (1 << 31) else 0
            gej = jnp.where(key >= pb + jnp.int32(step), 1.0, 0.0)
            cntj = jax.lax.dot_general(
                gej, ones_col, (((1,), (0,)), ((), ())),
                preferred_element_type=jnp.float32)
            jstar = jstar + jnp.where(cntj >= float(_K), 1.0, 0.0)
        p_pre = p_pre + (jstar.astype(jnp.int32) << t)
    x_thr = p_pre

    xb = jax.lax.broadcast_in_dim(x_thr, (_NP, _NP), (0, 1))
    gt = jnp.where(key > xb, 1.0, 0.0)
    eq = jnp.where(key == xb, 1.0, 0.0)
    g_row = jax.lax.dot_general(
        gt, ones_col, (((1,), (0,)), ((), ())),
        preferred_element_type=jnp.float32)
    upper = jnp.where(row_i < col_i, 1.0, 0.0)
    prefix = jax.lax.dot_general(
        eq, upper, (((1,), (0,)), ((), ())),
        preferred_element_type=jnp.float32)
    gb = jax.lax.broadcast_in_dim(g_row, (_NP, _NP), (0, 1))
    sel_eq = eq * jnp.where(gb + prefix < float(_K), 1.0, 0.0)
    rowvalid = jnp.where(row_i < _N, 1.0, 0.0)
    mask = (gt + sel_eq) * rowvalid

    mask_t = jax.lax.dot_general(
        mask, eye, (((0,), (0,)), ((), ())),
        preferred_element_type=jnp.float32)
    adj = jnp.clip(mask + mask_t, 0.0, 1.0)
    nmz = adj + eye * rowvalid * jnp.where(col_i < _N, 1.0, 0.0)
    common = jax.lax.dot_general(
        nmz, nmz, (((1,), (1,)), ((), ())),
        preferred_element_type=jnp.float32)
    maxc = jnp.max(common)
    denom = jnp.where(maxc > 0, maxc, 1.0)
    emask2 = adj * jnp.where(common > 1, 1.0, 0.0)
    structural = jnp.where(emask2 > 0, (common / denom) * common, 0.0)

    fused = (sim + structural) * mask
    deg = jnp.sum(fused, axis=0, keepdims=True)
    dp = deg ** -0.5
    dinv = jnp.where(jnp.isinf(dp), 0.0, dp)
    dinv_col = jax.lax.dot_general(
        eye, dinv, (((1,), (1,)), ((), ())),
        preferred_element_type=jnp.float32)
    wn = dinv_col * fused * dinv

    p2 = jax.lax.dot_general(
        y2, wn, (((1,), (0,)), ((), ())),
        preferred_element_type=jnp.float32)
    p3 = p2.reshape(_B, _S, _NP)
    bias_col = jax.lax.dot_general(
        eye[:_S, :_S], bias, (((1,), (1,)), ((), ())),
        preferred_element_type=jnp.float32)
    for b in range(_B):
        outb = jax.lax.dot_general(
            w, p3[b], (((0,), (0,)), ((), ())),
            preferred_element_type=jnp.float32)
        out_ref[b] = (outb + bias_col)[:, :_N]


def kernel(x, edge_index, weight, bias):
    del edge_index
    return pl.pallas_call(
        _fused_forward,
        out_shape=jax.ShapeDtypeStruct((_B, _S, _N), jnp.float32),
    )(x, weight, bias.reshape(1, _S))
